# baseline (speedup 1.0000x reference)
import dataclasses
import functools

import jax
import jax.numpy as jnp
from jax import lax
from jax.experimental import pallas as pl
from jax.experimental.pallas import tpu as pltpu
from jax.experimental.pallas import tpu_sc as plsc

VNUM = 10000
E = 320000
D = 128
EPS = 1e-7

NC = 2
NS = 16
L = 16
NW = NC * NS

CHUNK = 128
CPW = 81
ROWS = NW * CPW
EP = ROWS * CHUNK

ZC = 80
NZ = VNUM // ZC

_mesh = plsc.VectorSubcoreMesh(core_axis_name="c", subcore_axis_name="s")

_cp = pltpu.CompilerParams()
if "needs_layout_passes" in pltpu.CompilerParams.__dataclass_fields__:
    _cp = dataclasses.replace(_cp, needs_layout_passes=False)


@functools.partial(
    pl.kernel,
    out_type=jax.ShapeDtypeStruct((NC, VNUM, D), jnp.float32),
    mesh=_mesh,
    scratch_types=[
        pltpu.VMEM_SHARED((VNUM, D), jnp.float32),
        [pltpu.VMEM((CHUNK,), jnp.int32) for _ in range(3)],
        [pltpu.VMEM((CHUNK,), jnp.int32) for _ in range(3)],
        [pltpu.VMEM((CHUNK,), jnp.float32) for _ in range(3)],
        [pltpu.VMEM((CHUNK, D), jnp.float32) for _ in range(3)],
        [pltpu.SemaphoreType.DMA for _ in range(3)],
        [pltpu.SemaphoreType.DMA for _ in range(3)],
        [pltpu.SemaphoreType.DMA for _ in range(3)],
    ],
    compiler_params=_cp,
)
def _sc_graphconv(si_hbm, ti_hbm, w_hbm, vrepr_hbm, out_hbm, acc,
                  sis, tis, ws, gs, sems, ssems, msems):
    c = lax.axis_index("c")
    s = lax.axis_index("s")
    wid = s * NC + c
    r = gs[0]

    @pl.loop(0, ZC)
    def _(i):
        @pl.loop(0, D, step=L)
        def _(j):
            r[i, pl.ds(j, L)] = jnp.zeros((L,), jnp.float32)

    @pl.loop(0, (NZ + NS - 1) // NS)
    def _(k):
        j = k * NS + s

        @pl.when(j < NZ)
        def _():
            pltpu.sync_copy(r.at[pl.ds(0, ZC)],
                            acc.at[pl.ds(j * ZC, ZC)])

    plsc.subcore_barrier()

    base = wid * CPW

    def _issue(slot, row):
        off = row * CHUNK
        pltpu.sync_copy(si_hbm.at[pl.ds(off, CHUNK)], sis[slot])
        dg = pltpu.async_copy(vrepr_hbm.at[sis[slot]], gs[slot], sems[slot])
        dm = [pltpu.async_copy(ti_hbm.at[pl.ds(off, CHUNK)], tis[slot],
                               msems[slot]),
              pltpu.async_copy(w_hbm.at[pl.ds(off, CHUNK)], ws[slot],
                               msems[slot])]
        return dg, dm

    def _scale(slot):
        g = gs[slot]

        @plsc.parallel_loop(0, CHUNK, 1, unroll=8)
        def _(e):
            wv = plsc.load_gather(ws[slot], [jnp.full((L,), e, jnp.int32)])
            for j in range(0, D, L):
                g[e, pl.ds(j, L)] = g[e, pl.ds(j, L)] * wv

    def _scatter(slot):
        pltpu.async_copy(gs[slot], acc.at[tis[slot]], ssems[slot],
                         add=True)

    def _scatter_wait(slot):
        pltpu.make_async_copy(gs[slot], acc.at[tis[slot]],
                              ssems[slot]).wait()

    @pl.loop(0, CPW // 3)
    def _(k):
        row = base + 3 * k
        ds = []
        for j in range(3):
            @pl.when(k > 0)
            def _():
                _scatter_wait(j)

            ds.append(_issue(j, row + j))
        for j in range(3):
            dg, dm = ds[j]
            dg.wait()
            for d in dm:
                d.wait()
            _scale(j)
            _scatter(j)

    for j in range(3):
        _scatter_wait(j)

    plsc.subcore_barrier()

    @pl.loop(0, (NZ + NS - 1) // NS)
    def _(k):
        j = k * NS + s

        @pl.when(j < NZ)
        def _():
            pltpu.sync_copy(acc.at[pl.ds(j * ZC, ZC)],
                            out_hbm.at[c, pl.ds(j * ZC, ZC)])


def _tc_body(p_ref, lw_ref, lb_ref, sw_ref, sb_ref, loc_ref, std_ref):
    ptr = p_ref[0] + p_ref[1]
    dn = (((1,), (1,)), ((), ()))
    loc = lax.dot_general(ptr, lw_ref[...], dn,
                          preferred_element_type=jnp.float32,
                          precision=lax.Precision.HIGHEST)
    loc_ref[...] = loc + lb_ref[...]
    pre = lax.dot_general(ptr, sw_ref[...], dn,
                          preferred_element_type=jnp.float32,
                          precision=lax.Precision.HIGHEST)
    std_ref[...] = jax.nn.softplus(pre + sb_ref[...]) + EPS


_TCB = 1000


def _tc_project(partials, loc_W, loc_b, std_W, std_b):
    grid = (VNUM // _TCB,)
    return pl.pallas_call(
        _tc_body,
        grid=grid,
        in_specs=[
            pl.BlockSpec((NC, _TCB, D), lambda i: (0, i, 0)),
            pl.BlockSpec((D, D), lambda i: (0, 0)),
            pl.BlockSpec((1, D), lambda i: (0, 0)),
            pl.BlockSpec((D, D), lambda i: (0, 0)),
            pl.BlockSpec((1, D), lambda i: (0, 0)),
        ],
        out_specs=[
            pl.BlockSpec((_TCB, D), lambda i: (i, 0)),
            pl.BlockSpec((_TCB, D), lambda i: (i, 0)),
        ],
        out_shape=[
            jax.ShapeDtypeStruct((VNUM, D), jnp.float32),
            jax.ShapeDtypeStruct((VNUM, D), jnp.float32),
        ],
    )(partials, loc_W, loc_b, std_W, std_b)


def kernel(sidx, tidx, enorm, esgn, vrepr, loc_W, loc_b, std_W, std_b):
    pad = EP - E
    spread = jnp.arange(pad, dtype=jnp.int32) % VNUM
    si_p = jnp.concatenate([sidx.astype(jnp.int32), spread])
    ti_p = jnp.concatenate([tidx.astype(jnp.int32), spread])
    w_p = jnp.pad(esgn * enorm, (0, pad))

    partials = _sc_graphconv(si_p, ti_p, w_p, vrepr)
    loc, std = _tc_project(partials, loc_W, loc_b.reshape(1, D),
                           std_W, std_b.reshape(1, D))
    return (loc, std)


# speedup vs baseline: 1.0025x; 1.0025x over previous
import dataclasses
import functools

import jax
import jax.numpy as jnp
from jax import lax
from jax.experimental import pallas as pl
from jax.experimental.pallas import tpu as pltpu
from jax.experimental.pallas import tpu_sc as plsc

VNUM = 10000
E = 320000
D = 128
EPS = 1e-7

NC = 2
NS = 16
L = 16
NW = NC * NS

CHUNK = 128
CPW = 81
ROWS = NW * CPW
EP = ROWS * CHUNK

ZC = 80
NZ = VNUM // ZC

_mesh = plsc.VectorSubcoreMesh(core_axis_name="c", subcore_axis_name="s")

_cp = pltpu.CompilerParams()
if "needs_layout_passes" in pltpu.CompilerParams.__dataclass_fields__:
    _cp = dataclasses.replace(_cp, needs_layout_passes=False)


@functools.partial(
    pl.kernel,
    out_type=jax.ShapeDtypeStruct((NC, VNUM, D), jnp.float32),
    mesh=_mesh,
    scratch_types=[
        pltpu.VMEM_SHARED((VNUM, D), jnp.float32),
        [pltpu.VMEM((CHUNK,), jnp.int32) for _ in range(3)],
        [pltpu.VMEM((CHUNK,), jnp.int32) for _ in range(3)],
        [pltpu.VMEM((CHUNK,), jnp.float32) for _ in range(3)],
        [pltpu.VMEM((CHUNK, D), jnp.float32) for _ in range(3)],
        [pltpu.SemaphoreType.DMA for _ in range(3)],
        [pltpu.SemaphoreType.DMA for _ in range(3)],
        [pltpu.SemaphoreType.DMA for _ in range(3)],
    ],
    compiler_params=_cp,
)
def _sc_graphconv(si_hbm, ti_hbm, w_hbm, vrepr_hbm, out_hbm, acc,
                  sis, tis, ws, gs, sems, ssems, msems):
    c = lax.axis_index("c")
    s = lax.axis_index("s")
    wid = s * NC + c
    r = gs[0]

    @pl.loop(0, ZC)
    def _(i):
        @pl.loop(0, D, step=L)
        def _(j):
            r[i, pl.ds(j, L)] = jnp.zeros((L,), jnp.float32)

    @pl.loop(0, (NZ + NS - 1) // NS)
    def _(k):
        j = k * NS + s

        @pl.when(j < NZ)
        def _():
            pltpu.sync_copy(r.at[pl.ds(0, ZC)],
                            acc.at[pl.ds(j * ZC, ZC)])

    plsc.subcore_barrier()

    base = wid * CPW

    def _issue(slot, row):
        off = row * CHUNK
        pltpu.sync_copy(si_hbm.at[pl.ds(off, CHUNK)], sis[slot])
        dg = pltpu.async_copy(vrepr_hbm.at[sis[slot]], gs[slot], sems[slot])
        dm = [pltpu.async_copy(ti_hbm.at[pl.ds(off, CHUNK)], tis[slot],
                               msems[slot]),
              pltpu.async_copy(w_hbm.at[pl.ds(off, CHUNK)], ws[slot],
                               msems[slot])]
        return dg, dm

    def _scale(slot):
        g = gs[slot]

        @plsc.parallel_loop(0, CHUNK, 1, unroll=4)
        def _(e):
            wv = plsc.load_gather(ws[slot], [jnp.full((L,), e, jnp.int32)])
            for j in range(0, D, L):
                g[e, pl.ds(j, L)] = g[e, pl.ds(j, L)] * wv

    def _scatter(slot):
        pltpu.async_copy(gs[slot], acc.at[tis[slot]], ssems[slot],
                         add=True)

    def _scatter_wait(slot):
        pltpu.make_async_copy(gs[slot], acc.at[tis[slot]],
                              ssems[slot]).wait()

    @pl.loop(0, CPW // 3)
    def _(k):
        row = base + 3 * k
        ds = []
        for j in range(3):
            @pl.when(k > 0)
            def _():
                _scatter_wait(j)

            ds.append(_issue(j, row + j))
        for j in range(3):
            dg, dm = ds[j]
            dg.wait()
            for d in dm:
                d.wait()
            _scale(j)
            _scatter(j)

    for j in range(3):
        _scatter_wait(j)

    plsc.subcore_barrier()

    @pl.loop(0, (NZ + NS - 1) // NS)
    def _(k):
        j = k * NS + s

        @pl.when(j < NZ)
        def _():
            pltpu.sync_copy(acc.at[pl.ds(j * ZC, ZC)],
                            out_hbm.at[c, pl.ds(j * ZC, ZC)])


def _tc_body(p_ref, lw_ref, lb_ref, sw_ref, sb_ref, loc_ref, std_ref):
    ptr = p_ref[0] + p_ref[1]
    dn = (((1,), (1,)), ((), ()))
    loc = lax.dot_general(ptr, lw_ref[...], dn,
                          preferred_element_type=jnp.float32,
                          precision=lax.Precision.HIGHEST)
    loc_ref[...] = loc + lb_ref[...]
    pre = lax.dot_general(ptr, sw_ref[...], dn,
                          preferred_element_type=jnp.float32,
                          precision=lax.Precision.HIGHEST)
    std_ref[...] = jax.nn.softplus(pre + sb_ref[...]) + EPS


_TCB = 1000


def _tc_project(partials, loc_W, loc_b, std_W, std_b):
    grid = (VNUM // _TCB,)
    return pl.pallas_call(
        _tc_body,
        grid=grid,
        in_specs=[
            pl.BlockSpec((NC, _TCB, D), lambda i: (0, i, 0)),
            pl.BlockSpec((D, D), lambda i: (0, 0)),
            pl.BlockSpec((1, D), lambda i: (0, 0)),
            pl.BlockSpec((D, D), lambda i: (0, 0)),
            pl.BlockSpec((1, D), lambda i: (0, 0)),
        ],
        out_specs=[
            pl.BlockSpec((_TCB, D), lambda i: (i, 0)),
            pl.BlockSpec((_TCB, D), lambda i: (i, 0)),
        ],
        out_shape=[
            jax.ShapeDtypeStruct((VNUM, D), jnp.float32),
            jax.ShapeDtypeStruct((VNUM, D), jnp.float32),
        ],
    )(partials, loc_W, loc_b, std_W, std_b)


def kernel(sidx, tidx, enorm, esgn, vrepr, loc_W, loc_b, std_W, std_b):
    pad = EP - E
    spread = jnp.arange(pad, dtype=jnp.int32) % VNUM
    si_p = jnp.concatenate([sidx.astype(jnp.int32), spread])
    ti_p = jnp.concatenate([tidx.astype(jnp.int32), spread])
    w_p = jnp.pad(esgn * enorm, (0, pad))

    partials = _sc_graphconv(si_p, ti_p, w_p, vrepr)
    loc, std = _tc_project(partials, loc_W, loc_b.reshape(1, D),
                           std_W, std_b.reshape(1, D))
    return (loc, std)
